# baseline (speedup 1.0000x reference)
import functools

import jax
import jax.numpy as jnp
from jax.experimental import pallas as pl
from jax.experimental.pallas import tpu as pltpu


def _dcn_loss_kernel(e2_ref, c2_ref, out_ref, *, inv_n):
    i = pl.program_id(0)
    e2 = e2_ref[...]
    c2 = c2_ref[...]
    lane_e = jax.lax.broadcasted_iota(jnp.int32, e2.shape, 1) < 64
    lane_c = jax.lax.broadcasted_iota(jnp.int32, c2.shape, 1) < 64
    c2r = jnp.concatenate([c2[:, 64:], c2[:, :64]], axis=1)
    cfe = jnp.concatenate([c2, c2r], axis=0)
    cfo = jnp.concatenate([c2r, c2], axis=0)
    cfe_bf = (-2.0 * cfe).astype(jnp.bfloat16)
    cfo_bf = (-2.0 * cfo).astype(jnp.bfloat16)
    qe = jnp.where(jnp.concatenate([lane_c, lane_c], axis=0), cfe, 0.0)
    qe_bf = (qe * qe).astype(jnp.bfloat16)
    ones_bf = jnp.ones((8, 128), jnp.bfloat16)
    b2 = jax.lax.dot_general(
        ones_bf, qe_bf, (((1,), (1,)), ((), ())),
        preferred_element_type=jnp.float32)[0:1]
    e_lo = jnp.where(lane_e, e2, 0.0).astype(jnp.bfloat16)
    e_hi = jnp.where(lane_e, 0.0, e2).astype(jnp.bfloat16)
    de = jax.lax.dot_general(
        e_lo, cfe_bf, (((1,), (1,)), ((), ())),
        preferred_element_type=jnp.float32)
    do = jax.lax.dot_general(
        e_hi, cfo_bf, (((1,), (1,)), ((), ())),
        preferred_element_type=jnp.float32)
    me = jnp.min(de + b2, axis=1)
    mo = jnp.min(do + b2, axis=1)
    sq = e2 * e2
    a2e = jnp.sum(jnp.where(lane_e, sq, 0.0), axis=1)
    a2o = jnp.sum(jnp.where(lane_e, 0.0, sq), axis=1)
    tot = (jnp.sum(jnp.maximum(a2e + me, 0.0)) +
           jnp.sum(jnp.maximum(a2o + mo, 0.0)))

    @pl.when(i == 0)
    def _init():
        out_ref[0, 0] = 0.0

    out_ref[0, 0] += tot * inv_n


def kernel(embedded, centers):
    n, d = embedded.shape
    k, _ = centers.shape
    e2 = embedded.reshape(n // 2, 2 * d)
    c2 = centers.reshape(k // 2, 2 * d)
    n2 = n // 2
    bn2 = 2048 if n2 % 2048 == 0 else n2
    grid = (n2 // bn2,)
    out = pl.pallas_call(
        functools.partial(_dcn_loss_kernel, inv_n=1.0 / n),
        grid=grid,
        in_specs=[
            pl.BlockSpec((bn2, 2 * d), lambda i: (i, 0)),
            pl.BlockSpec((k // 2, 2 * d), lambda i: (0, 0)),
        ],
        out_specs=pl.BlockSpec(memory_space=pltpu.SMEM),
        out_shape=jax.ShapeDtypeStruct((1, 1), jnp.float32),
    )(e2, c2)
    return out[0, 0]


# speedup vs baseline: 1.1704x; 1.1704x over previous
import functools

import jax
import jax.numpy as jnp
from jax.experimental import pallas as pl
from jax.experimental.pallas import tpu as pltpu


def _dcn_loss_kernel(e2_ref, c2_ref, out_ref, *, inv_n):
    i = pl.program_id(0)
    e2 = e2_ref[...]
    c2 = c2_ref[...]
    lane_e = jax.lax.broadcasted_iota(jnp.int32, e2.shape, 1) < 64
    lane_c = jax.lax.broadcasted_iota(jnp.int32, c2.shape, 1) < 64
    c2r = jnp.concatenate([c2[:, 64:], c2[:, :64]], axis=1)
    cfe = jnp.concatenate([c2, c2r], axis=0)
    cfo = jnp.concatenate([c2r, c2], axis=0)
    cfe_bf = (-2.0 * cfe).astype(jnp.bfloat16)
    cfo_bf = (-2.0 * cfo).astype(jnp.bfloat16)
    qe = jnp.where(jnp.concatenate([lane_c, lane_c], axis=0), cfe, 0.0)
    qe_bf = (qe * qe).astype(jnp.bfloat16)
    ones_bf = jnp.ones((8, 128), jnp.bfloat16)
    b2 = jax.lax.dot_general(
        ones_bf, qe_bf, (((1,), (1,)), ((), ())),
        preferred_element_type=jnp.float32)[0:1]
    zero_bf = jnp.zeros((), jnp.bfloat16)
    e_lo = jnp.where(lane_e, e2, zero_bf)
    e_hi = jnp.where(lane_e, zero_bf, e2)
    de = jax.lax.dot_general(
        e_lo, cfe_bf, (((1,), (1,)), ((), ())),
        preferred_element_type=jnp.float32)
    do = jax.lax.dot_general(
        e_hi, cfo_bf, (((1,), (1,)), ((), ())),
        preferred_element_type=jnp.float32)
    me = jnp.min(de + b2, axis=1)
    mo = jnp.min(do + b2, axis=1)
    e2f = e2.astype(jnp.float32)
    sq = e2f * e2f
    a2e = jnp.sum(jnp.where(lane_e, sq, 0.0), axis=1)
    a2o = jnp.sum(jnp.where(lane_e, 0.0, sq), axis=1)
    tot = (jnp.sum(jnp.maximum(a2e + me, 0.0)) +
           jnp.sum(jnp.maximum(a2o + mo, 0.0)))

    @pl.when(i == 0)
    def _init():
        out_ref[0, 0] = 0.0

    out_ref[0, 0] += tot * inv_n


def kernel(embedded, centers):
    n, d = embedded.shape
    k, _ = centers.shape
    e2 = embedded.astype(jnp.bfloat16).reshape(n // 2, 2 * d)
    c2 = centers.reshape(k // 2, 2 * d)
    n2 = n // 2
    bn2 = 2048 if n2 % 2048 == 0 else n2
    grid = (n2 // bn2,)
    out = pl.pallas_call(
        functools.partial(_dcn_loss_kernel, inv_n=1.0 / n),
        grid=grid,
        in_specs=[
            pl.BlockSpec((bn2, 2 * d), lambda i: (i, 0)),
            pl.BlockSpec((k // 2, 2 * d), lambda i: (0, 0)),
        ],
        out_specs=pl.BlockSpec(memory_space=pltpu.SMEM),
        out_shape=jax.ShapeDtypeStruct((1, 1), jnp.float32),
    )(e2, c2)
    return out[0, 0]


# speedup vs baseline: 1.9448x; 1.6617x over previous
import functools

import jax
import jax.numpy as jnp
from jax.experimental import pallas as pl
from jax.experimental.pallas import tpu as pltpu


def _dcn_loss_kernel(et_ref, ct_ref, out_ref, *, inv_n):
    i = pl.program_id(0)
    et = et_ref[...]
    ct = ct_ref[...]
    et_bf = et.astype(jnp.bfloat16)
    ctm2_bf = (-2.0 * ct).astype(jnp.bfloat16)
    dots = jax.lax.dot_general(
        ctm2_bf, et_bf, (((0,), (0,)), ((), ())),
        preferred_element_type=jnp.float32)
    csq = ct * ct
    b2 = jax.lax.dot_general(
        csq, jnp.ones((csq.shape[0], 8), jnp.float32),
        (((0,), (0,)), ((), ())),
        preferred_element_type=jnp.float32)[:, 0:1]
    m = jnp.min(dots + b2, axis=0)
    a2 = jnp.sum(et * et, axis=0)
    tot = jnp.sum(jnp.maximum(a2 + m, 0.0))

    @pl.when(i == 0)
    def _init():
        out_ref[0, 0] = 0.0

    out_ref[0, 0] += tot * inv_n


def kernel(embedded, centers):
    n, d = embedded.shape
    k, _ = centers.shape
    et = embedded.T
    ct = centers.T
    bn = 2048 if n % 2048 == 0 else n
    grid = (n // bn,)
    out = pl.pallas_call(
        functools.partial(_dcn_loss_kernel, inv_n=1.0 / n),
        grid=grid,
        in_specs=[
            pl.BlockSpec((d, bn), lambda i: (0, i)),
            pl.BlockSpec((d, k), lambda i: (0, 0)),
        ],
        out_specs=pl.BlockSpec(memory_space=pltpu.SMEM),
        out_shape=jax.ShapeDtypeStruct((1, 1), jnp.float32),
    )(et, ct)
    return out[0, 0]


# speedup vs baseline: 2.0985x; 1.0790x over previous
import functools

import jax
import jax.numpy as jnp
from jax.experimental import pallas as pl
from jax.experimental.pallas import tpu as pltpu


def _dcn_loss_kernel(et_ref, ct_ref, out_ref, *, inv_n):
    i = pl.program_id(0)
    et = et_ref[...]
    ct = ct_ref[...]
    et_bf = et.astype(jnp.bfloat16)
    ctm2_bf = (-2.0 * ct).astype(jnp.bfloat16)
    b2 = jnp.sum(ct * ct, axis=0, keepdims=True)
    b2_hi = b2.astype(jnp.bfloat16)
    b2_lo = (b2 - b2_hi.astype(jnp.float32)).astype(jnp.bfloat16)
    ct_aug = jnp.concatenate([ctm2_bf, b2_hi, b2_lo], axis=0)
    ones2 = jnp.ones((2, et.shape[1]), jnp.bfloat16)
    et_aug = jnp.concatenate([et_bf, ones2], axis=0)
    dist0 = jax.lax.dot_general(
        ct_aug, et_aug, (((0,), (0,)), ((), ())),
        preferred_element_type=jnp.float32)
    m = jnp.min(dist0, axis=0)
    a2 = jnp.sum(et * et, axis=0)
    tot = jnp.sum(jnp.maximum(a2 + m, 0.0))

    @pl.when(i == 0)
    def _init():
        out_ref[0, 0] = 0.0

    out_ref[0, 0] += tot * inv_n


def kernel(embedded, centers):
    n, d = embedded.shape
    k, _ = centers.shape
    et = embedded.T
    ct = centers.T
    bn = 2048 if n % 2048 == 0 else n
    grid = (n // bn,)
    out = pl.pallas_call(
        functools.partial(_dcn_loss_kernel, inv_n=1.0 / n),
        grid=grid,
        in_specs=[
            pl.BlockSpec((d, bn), lambda i: (0, i)),
            pl.BlockSpec((d, k), lambda i: (0, 0)),
        ],
        out_specs=pl.BlockSpec(memory_space=pltpu.SMEM),
        out_shape=jax.ShapeDtypeStruct((1, 1), jnp.float32),
    )(et, ct)
    return out[0, 0]


# speedup vs baseline: 2.3388x; 1.1145x over previous
import functools

import jax
import jax.numpy as jnp
from jax.experimental import pallas as pl
from jax.experimental.pallas import tpu as pltpu


def _dcn_loss_kernel(et_ref, ct_ref, out_ref, *, inv_n):
    i = pl.program_id(0)
    et = et_ref[...]
    ct = ct_ref[...]
    et_bf = et.astype(jnp.bfloat16)
    ctm2_bf = (-2.0 * ct).astype(jnp.bfloat16)
    b2 = jnp.sum(ct * ct, axis=0, keepdims=True)
    b2_hi = b2.astype(jnp.bfloat16)
    b2_lo = (b2 - b2_hi.astype(jnp.float32)).astype(jnp.bfloat16)
    ct_aug = jnp.concatenate([ctm2_bf, b2_hi, b2_lo], axis=0)
    ones2 = jnp.ones((2, et.shape[1]), jnp.bfloat16)
    et_aug = jnp.concatenate([et_bf, ones2], axis=0)
    dist0 = jax.lax.dot_general(
        ct_aug, et_aug, (((0,), (0,)), ((), ())),
        preferred_element_type=jnp.float32)
    m = jnp.min(dist0, axis=0)
    a2 = jnp.sum(et * et, axis=0)
    tot = jnp.sum(jnp.maximum(a2 + m, 0.0))

    @pl.when(i == 0)
    def _init():
        out_ref[0, 0] = 0.0

    out_ref[0, 0] += tot * inv_n


def kernel(embedded, centers):
    n, d = embedded.shape
    k, _ = centers.shape
    et = embedded.T
    ct = centers.T
    bn = 4096 if n % 4096 == 0 else n
    grid = (n // bn,)
    out = pl.pallas_call(
        functools.partial(_dcn_loss_kernel, inv_n=1.0 / n),
        grid=grid,
        in_specs=[
            pl.BlockSpec((d, bn), lambda i: (0, i)),
            pl.BlockSpec((d, k), lambda i: (0, 0)),
        ],
        out_specs=pl.BlockSpec(memory_space=pltpu.SMEM),
        out_shape=jax.ShapeDtypeStruct((1, 1), jnp.float32),
    )(et, ct)
    return out[0, 0]


# speedup vs baseline: 2.3742x; 1.0152x over previous
import functools

import jax
import jax.numpy as jnp
from jax.experimental import pallas as pl
from jax.experimental.pallas import tpu as pltpu


def _dcn_loss_kernel(et_ref, ct_ref, out_ref, *, inv_n):
    i = pl.program_id(0)
    et = et_ref[...]
    ct = ct_ref[...]
    et_bf = et.astype(jnp.bfloat16)
    ctm2_bf = (-2.0 * ct).astype(jnp.bfloat16)
    b2 = jnp.sum(ct * ct, axis=0, keepdims=True)
    b2_hi = b2.astype(jnp.bfloat16)
    b2_lo = (b2 - b2_hi.astype(jnp.float32)).astype(jnp.bfloat16)
    ct_aug = jnp.concatenate([ctm2_bf, b2_hi, b2_lo], axis=0)
    ones2 = jnp.ones((2, et.shape[1]), jnp.bfloat16)
    et_aug = jnp.concatenate([et_bf, ones2], axis=0)
    dist0 = jax.lax.dot_general(
        ct_aug, et_aug, (((0,), (0,)), ((), ())),
        preferred_element_type=jnp.float32)
    m = jnp.min(dist0, axis=0)
    a2 = jnp.sum(et * et, axis=0)
    tot = jnp.sum(jnp.maximum(a2 + m, 0.0))

    @pl.when(i == 0)
    def _init():
        out_ref[0, 0] = 0.0

    out_ref[0, 0] += tot * inv_n


def kernel(embedded, centers):
    n, d = embedded.shape
    k, _ = centers.shape
    et = embedded.T
    ct = centers.T
    bn = 8192 if n % 8192 == 0 else n
    grid = (n // bn,)
    out = pl.pallas_call(
        functools.partial(_dcn_loss_kernel, inv_n=1.0 / n),
        grid=grid,
        in_specs=[
            pl.BlockSpec((d, bn), lambda i: (0, i)),
            pl.BlockSpec((d, k), lambda i: (0, 0)),
        ],
        out_specs=pl.BlockSpec(memory_space=pltpu.SMEM),
        out_shape=jax.ShapeDtypeStruct((1, 1), jnp.float32),
    )(et, ct)
    return out[0, 0]
